# baseline (speedup 1.0000x reference)
import functools

import jax
import jax.numpy as jnp
from jax import lax
from jax.experimental import pallas as pl
from jax.experimental.pallas import tpu as pltpu
from jax.experimental.pallas import tpu_sc as plsc

N = 10000
F = 128
NC = 2
NS = 16
L = 16
NW = NC * NS

NPAD = 10240
RPT = NPAD // NS

E = 320000
CHUNK = 128
NCHUNK = 80
EPT = NCHUNK * CHUNK
EPAD = EPT * NW

BM = 2000
GRID = N // BM

_sc_mesh = plsc.VectorSubcoreMesh(
    core_axis_name="c", subcore_axis_name="s", num_cores=NC, num_subcores=NS)


@functools.partial(
    pl.kernel,
    out_type=jax.ShapeDtypeStruct((NC * NPAD,), jnp.float32),
    mesh=_sc_mesh,
    scratch_types=[
        pltpu.VMEM((2 * NCHUNK, CHUNK // 2), jnp.int32),
        pltpu.VMEM((CHUNK // 2,), jnp.float32),
        pltpu.VMEM((RPT,), jnp.float32),
        pltpu.VMEM_SHARED((NPAD,), jnp.float32),
        pltpu.SemaphoreType.DMA,
    ],
)
def _sc_deg(dst_hbm, out_hbm, dstv, ones_v, zv, deg_sh, dsem):
    c = lax.axis_index("c")
    s = lax.axis_index("s")
    wid = c * NS + s

    pltpu.sync_copy(dst_hbm.at[wid], dstv)
    for i in range(RPT // L):
        zv[pl.ds(i * L, L)] = jnp.zeros((L,), jnp.float32)
    for i in range(CHUNK // 2 // L):
        ones_v[pl.ds(i * L, L)] = jnp.ones((L,), jnp.float32)

    pltpu.sync_copy(zv, deg_sh.at[pl.ds(s * RPT, RPT)])
    plsc.subcore_barrier()

    def body(j, carry):
        pltpu.async_copy(ones_v, deg_sh.at[dstv.at[j]], dsem, add=True)
        return carry

    lax.fori_loop(0, 2 * NCHUNK, body, 0)

    def drain(j, carry):
        pltpu.make_async_copy(ones_v, deg_sh.at[dstv.at[0]], dsem).wait()
        return carry

    lax.fori_loop(0, 2 * NCHUNK, drain, 0)
    plsc.subcore_barrier()
    pltpu.sync_copy(deg_sh.at[pl.ds(s * RPT, RPT)], zv)
    pltpu.sync_copy(zv,
                    out_hbm.at[pl.ds(c * NPAD + s * RPT, RPT)])


ZR = 128
HCHUNK = 40


@functools.partial(
    pl.kernel,
    out_type=jax.ShapeDtypeStruct((NC, NPAD, F), jnp.float32),
    mesh=_sc_mesh,
    scratch_types=[
        pltpu.VMEM((HCHUNK, CHUNK // 2), jnp.int32),
        pltpu.VMEM((HCHUNK, CHUNK // 2), jnp.int32),
        pltpu.VMEM((CHUNK, F), jnp.float32),
        pltpu.VMEM((CHUNK, F), jnp.float32),
        pltpu.VMEM_SHARED((NPAD, F), jnp.float32),
        [pltpu.SemaphoreType.DMA] * 4,
        [pltpu.SemaphoreType.DMA] * 4,
    ],
)
def _sc_agg(h_hbm, src_hbm, dst_hbm, zeros_hbm, out_hbm,
            srcv, dstv, rb0, rb1, acc_sh, gsems, ssems):
    c = lax.axis_index("c")
    s = lax.axis_index("s")
    wid = c * NS + s

    pltpu.sync_copy(zeros_hbm, rb0)
    for i in range(RPT // ZR):
        pltpu.async_copy(rb0, acc_sh.at[pl.ds(s * RPT + i * ZR, ZR)], ssems[0])

    C2 = CHUNK // 2
    NCH2 = HCHUNK
    NJJ = NCH2 // 4

    def slot(k):
        rb = (rb0, rb0, rb1, rb1)[k]
        return rb.at[pl.ds((k % 2) * C2, C2)]

    for p in range(4):
        pltpu.sync_copy(src_hbm.at[wid, pl.ds(p * NCH2, NCH2)], srcv)
        pltpu.sync_copy(dst_hbm.at[wid, pl.ds(p * NCH2, NCH2)], dstv)
        if p == 0:
            for i in range(RPT // ZR):
                pltpu.make_async_copy(
                    rb0, acc_sh.at[pl.ds(s * RPT + i * ZR, ZR)],
                    ssems[0]).wait()
            plsc.subcore_barrier()

        pltpu.async_copy(h_hbm.at[srcv.at[0]], slot(0), gsems[0])
        pltpu.async_copy(h_hbm.at[srcv.at[1]], slot(1), gsems[1])

        def body(jj, carry):
            for k in range(4):
                j = jj * 4 + k
                m = (k + 2) % 4
                pltpu.make_async_copy(
                    h_hbm.at[srcv.at[0]], slot(k), gsems[k]).wait()
                pltpu.async_copy(
                    slot(k), acc_sh.at[dstv.at[j]], ssems[k], add=True)
                if k < 2:
                    @pl.when(jj >= 1)
                    def _():
                        pltpu.make_async_copy(
                            slot(m), acc_sh.at[dstv.at[0]], ssems[m]).wait()
                    pltpu.async_copy(h_hbm.at[srcv.at[j + 2]], slot(m),
                                     gsems[m])
                else:
                    @pl.when(jj < NJJ - 1)
                    def _():
                        pltpu.make_async_copy(
                            slot(m), acc_sh.at[dstv.at[0]], ssems[m]).wait()
                        pltpu.async_copy(h_hbm.at[srcv.at[j + 2]], slot(m),
                                         gsems[m])
            return carry

        lax.fori_loop(0, NJJ, body, 0)
        for k in range(4):
            pltpu.make_async_copy(
                slot(k), acc_sh.at[dstv.at[0]], ssems[k]).wait()

    plsc.subcore_barrier()
    NRO = RPT // ZR
    bufs = (rb0, rb1)
    isems = (gsems[0], gsems[1])
    osems = (ssems[0], ssems[1])
    pltpu.async_copy(acc_sh.at[pl.ds(s * RPT, ZR)], rb0, gsems[0])
    for i in range(NRO):
        b = i % 2
        pltpu.make_async_copy(
            acc_sh.at[pl.ds(s * RPT, ZR)], bufs[b], isems[b]).wait()
        if i + 1 < NRO:
            nb = (i + 1) % 2
            if i >= 1:
                pltpu.make_async_copy(
                    bufs[nb], out_hbm.at[c, pl.ds(s * RPT, ZR)],
                    osems[nb]).wait()
            pltpu.async_copy(
                acc_sh.at[pl.ds(s * RPT + (i + 1) * ZR, ZR)], bufs[nb],
                isems[nb])
        pltpu.async_copy(
            bufs[b], out_hbm.at[c, pl.ds(s * RPT + i * ZR, ZR)], osems[b])
    pltpu.make_async_copy(
        rb0, out_hbm.at[c, pl.ds(s * RPT, ZR)], ssems[0]).wait()
    pltpu.make_async_copy(
        rb1, out_hbm.at[c, pl.ds(s * RPT, ZR)], ssems[1]).wait()


def _mm_scale_body(x_ref, w_ref, degp_ref, hs_ref, dinv_ref):
    h = jnp.dot(x_ref[...], w_ref[...], preferred_element_type=jnp.float32)
    dp = degp_ref[...]
    deg = dp[:, 0] + dp[:, 1] + 1.0
    dinv = lax.rsqrt(deg)
    hs_ref[...] = h * dinv[:, None]
    dinv_ref[...] = dinv[:, None]


_mm_scale = pl.pallas_call(
    _mm_scale_body,
    grid=(GRID,),
    in_specs=[
        pl.BlockSpec((BM, F), lambda i: (i, 0)),
        pl.BlockSpec((F, F), lambda i: (0, 0)),
        pl.BlockSpec((BM, NC), lambda i: (i, 0)),
    ],
    out_specs=[
        pl.BlockSpec((BM, F), lambda i: (i, 0)),
        pl.BlockSpec((BM, 1), lambda i: (i, 0)),
    ],
    out_shape=[
        jax.ShapeDtypeStruct((N, F), jnp.float32),
        jax.ShapeDtypeStruct((N, 1), jnp.float32),
    ],
)


def _fuse1_body(p_ref, hs1_ref, dinv_ref, b1_ref, w2_ref, h2s_ref):
    dinv = dinv_ref[...]
    t = (p_ref[0] + p_ref[1] + hs1_ref[...]) * dinv + b1_ref[...]
    t = jnp.maximum(t, 0.0)
    h2 = jnp.dot(t, w2_ref[...], preferred_element_type=jnp.float32)
    h2s_ref[...] = h2 * dinv


_fuse1 = pl.pallas_call(
    _fuse1_body,
    grid=(GRID,),
    in_specs=[
        pl.BlockSpec((NC, BM, F), lambda i: (0, i, 0)),
        pl.BlockSpec((BM, F), lambda i: (i, 0)),
        pl.BlockSpec((BM, 1), lambda i: (i, 0)),
        pl.BlockSpec((F,), lambda i: (0,)),
        pl.BlockSpec((F, F), lambda i: (0, 0)),
    ],
    out_specs=pl.BlockSpec((BM, F), lambda i: (i, 0)),
    out_shape=jax.ShapeDtypeStruct((N, F), jnp.float32),
)


def _fuse2_body(q_ref, h2s_ref, dinv_ref, b2_ref, wc_ref, bc_ref, out_ref):
    dinv = dinv_ref[...]
    t = (q_ref[0] + q_ref[1] + h2s_ref[...]) * dinv + b2_ref[...]
    t = jnp.maximum(t, 0.0)
    logits = jnp.dot(t, wc_ref[...], preferred_element_type=jnp.float32)
    logits = logits + bc_ref[...]
    m = jnp.max(logits, axis=1, keepdims=True)
    e = jnp.exp(logits - m)
    lse = jnp.log(jnp.sum(e, axis=1, keepdims=True)) + m
    out_ref[...] = logits - lse


_fuse2 = pl.pallas_call(
    _fuse2_body,
    grid=(GRID,),
    in_specs=[
        pl.BlockSpec((NC, BM, F), lambda i: (0, i, 0)),
        pl.BlockSpec((BM, F), lambda i: (i, 0)),
        pl.BlockSpec((BM, 1), lambda i: (i, 0)),
        pl.BlockSpec((F,), lambda i: (0,)),
        pl.BlockSpec((F, F), lambda i: (0, 0)),
        pl.BlockSpec((F,), lambda i: (0,)),
    ],
    out_specs=pl.BlockSpec((BM, F), lambda i: (i, 0)),
    out_shape=jax.ShapeDtypeStruct((N, F), jnp.float32),
)


@jax.jit
def _run(x, edge_index, W1, b1, W2, b2, Wc, bc):
    src = edge_index[0].astype(jnp.int32)
    dst = edge_index[1].astype(jnp.int32)
    padn = EPAD - E
    ar = jnp.arange(padn, dtype=jnp.int32)
    pad_src = (ar * 131) % N
    pad_dst = N + ar % (NPAD - N)
    srcp = jnp.concatenate([src, pad_src]).reshape(NW, 2 * NCHUNK, CHUNK // 2)
    dstp = jnp.concatenate([dst, pad_dst]).reshape(NW, 2 * NCHUNK, CHUNK // 2)
    zrows = jnp.zeros((ZR, F), jnp.float32)

    degp = _sc_deg(dstp).reshape(NC, NPAD)[:, :N].T
    hs1, dinv = _mm_scale(x, W1, degp)
    agg1 = _sc_agg(hs1, srcp, dstp, zrows)
    h2s = _fuse1(agg1, hs1, dinv, b1, W2)
    agg2 = _sc_agg(h2s, srcp, dstp, zrows)
    return _fuse2(agg2, h2s, dinv, b2, Wc, bc)


def kernel(x, edge_index, W1, b1, W2, b2, Wc, bc):
    return _run(x, edge_index, W1, b1, W2, b2, Wc, bc)


# speedup vs baseline: 1.1276x; 1.1276x over previous
import functools

import jax
import jax.numpy as jnp
from jax import lax
from jax.experimental import pallas as pl
from jax.experimental.pallas import tpu as pltpu
from jax.experimental.pallas import tpu_sc as plsc

N = 10000
F = 128
NC = 2
NS = 16
L = 16
NW = NC * NS

NPAD = 10240
RPT = NPAD // NS

E = 320000
CHUNK = 128
NCHUNK = 80
EPT = NCHUNK * CHUNK
EPAD = EPT * NW

BM = 2000
GRID = N // BM

_sc_mesh = plsc.VectorSubcoreMesh(
    core_axis_name="c", subcore_axis_name="s", num_cores=NC, num_subcores=NS)


@functools.partial(
    pl.kernel,
    out_type=jax.ShapeDtypeStruct((NC * NPAD,), jnp.float32),
    mesh=_sc_mesh,
    scratch_types=[
        pltpu.VMEM((NCHUNK, CHUNK), jnp.int32),
        pltpu.VMEM((CHUNK,), jnp.float32),
        pltpu.VMEM((RPT,), jnp.float32),
        pltpu.VMEM_SHARED((NPAD,), jnp.float32),
        pltpu.SemaphoreType.DMA,
    ],
)
def _sc_deg(dst_hbm, out_hbm, dstv, ones_v, zv, deg_sh, dsem):
    c = lax.axis_index("c")
    s = lax.axis_index("s")
    wid = c * NS + s

    pltpu.sync_copy(dst_hbm.at[wid], dstv)
    for i in range(RPT // L):
        zv[pl.ds(i * L, L)] = jnp.zeros((L,), jnp.float32)
    for i in range(CHUNK // L):
        ones_v[pl.ds(i * L, L)] = jnp.ones((L,), jnp.float32)

    pltpu.sync_copy(zv, deg_sh.at[pl.ds(s * RPT, RPT)])
    plsc.subcore_barrier()

    def body(j, carry):
        pltpu.async_copy(ones_v, deg_sh.at[dstv.at[j]], dsem, add=True)
        return carry

    lax.fori_loop(0, NCHUNK, body, 0)

    def drain(j, carry):
        pltpu.make_async_copy(ones_v, deg_sh.at[dstv.at[0]], dsem).wait()
        return carry

    lax.fori_loop(0, NCHUNK, drain, 0)
    plsc.subcore_barrier()
    pltpu.sync_copy(deg_sh.at[pl.ds(s * RPT, RPT)], zv)
    pltpu.sync_copy(zv,
                    out_hbm.at[pl.ds(c * NPAD + s * RPT, RPT)])


ZR = 128
HCHUNK = 40


@functools.partial(
    pl.kernel,
    out_type=jax.ShapeDtypeStruct((NC, NPAD, F), jnp.float32),
    mesh=_sc_mesh,
    scratch_types=[
        pltpu.VMEM((HCHUNK, CHUNK), jnp.int32),
        pltpu.VMEM((HCHUNK, CHUNK), jnp.int32),
        pltpu.VMEM((CHUNK, F), jnp.float32),
        pltpu.VMEM((CHUNK, F), jnp.float32),
        pltpu.VMEM_SHARED((NPAD, F), jnp.float32),
        pltpu.SemaphoreType.DMA,
        pltpu.SemaphoreType.DMA,
        pltpu.SemaphoreType.DMA,
        pltpu.SemaphoreType.DMA,
    ],
)
def _sc_agg(h_hbm, src_hbm, dst_hbm, zeros_hbm, out_hbm,
            srcv, dstv, rb0, rb1, acc_sh,
            gsem0, gsem1, ssem0, ssem1):
    c = lax.axis_index("c")
    s = lax.axis_index("s")
    wid = c * NS + s

    pltpu.sync_copy(zeros_hbm, rb0)
    for i in range(RPT // ZR):
        pltpu.async_copy(rb0, acc_sh.at[pl.ds(s * RPT + i * ZR, ZR)], ssem0)

    for p in range(2):
        pltpu.sync_copy(src_hbm.at[wid, pl.ds(p * HCHUNK, HCHUNK)], srcv)
        pltpu.sync_copy(dst_hbm.at[wid, pl.ds(p * HCHUNK, HCHUNK)], dstv)
        if p == 0:
            for i in range(RPT // ZR):
                pltpu.make_async_copy(
                    rb0, acc_sh.at[pl.ds(s * RPT + i * ZR, ZR)], ssem0).wait()
            plsc.subcore_barrier()

        pltpu.async_copy(h_hbm.at[srcv.at[0]], rb0, gsem0)
        pltpu.async_copy(h_hbm.at[srcv.at[1]], rb1, gsem1)

        def body(jj, carry):
            j0 = jj * 2
            last = HCHUNK // 2 - 1

            pltpu.make_async_copy(h_hbm.at[srcv.at[0]], rb0, gsem0).wait()
            pltpu.async_copy(rb0, acc_sh.at[dstv.at[j0]], ssem0, add=True)

            @pl.when(jj < last)
            def _():
                pltpu.make_async_copy(rb0, acc_sh.at[dstv.at[0]], ssem0).wait()
                pltpu.async_copy(h_hbm.at[srcv.at[j0 + 2]], rb0, gsem0)

            pltpu.make_async_copy(h_hbm.at[srcv.at[0]], rb1, gsem1).wait()
            pltpu.async_copy(rb1, acc_sh.at[dstv.at[j0 + 1]], ssem1, add=True)

            @pl.when(jj < last)
            def _():
                pltpu.make_async_copy(rb1, acc_sh.at[dstv.at[0]], ssem1).wait()
                pltpu.async_copy(h_hbm.at[srcv.at[j0 + 3]], rb1, gsem1)

            return carry

        lax.fori_loop(0, HCHUNK // 2, body, 0)
        pltpu.make_async_copy(rb0, acc_sh.at[dstv.at[0]], ssem0).wait()
        pltpu.make_async_copy(rb1, acc_sh.at[dstv.at[0]], ssem1).wait()

    plsc.subcore_barrier()
    NRO = RPT // ZR
    bufs = (rb0, rb1)
    isems = (gsem0, gsem1)
    osems = (ssem0, ssem1)
    pltpu.async_copy(acc_sh.at[pl.ds(s * RPT, ZR)], rb0, gsem0)
    for i in range(NRO):
        b = i % 2
        pltpu.make_async_copy(
            acc_sh.at[pl.ds(s * RPT, ZR)], bufs[b], isems[b]).wait()
        if i + 1 < NRO:
            nb = (i + 1) % 2
            if i >= 1:
                pltpu.make_async_copy(
                    bufs[nb], out_hbm.at[c, pl.ds(s * RPT, ZR)],
                    osems[nb]).wait()
            pltpu.async_copy(
                acc_sh.at[pl.ds(s * RPT + (i + 1) * ZR, ZR)], bufs[nb],
                isems[nb])
        pltpu.async_copy(
            bufs[b], out_hbm.at[c, pl.ds(s * RPT + i * ZR, ZR)], osems[b])
    pltpu.make_async_copy(
        rb0, out_hbm.at[c, pl.ds(s * RPT, ZR)], ssem0).wait()
    pltpu.make_async_copy(
        rb1, out_hbm.at[c, pl.ds(s * RPT, ZR)], ssem1).wait()


def _mm_scale_body(x_ref, w_ref, degp_ref, hs_ref, dinv_ref):
    h = jnp.dot(x_ref[...], w_ref[...], preferred_element_type=jnp.float32)
    dp = degp_ref[...]
    deg = dp[:, 0] + dp[:, 1] + 1.0
    dinv = lax.rsqrt(deg)
    hs_ref[...] = h * dinv[:, None]
    dinv_ref[...] = dinv[:, None]


_mm_scale = pl.pallas_call(
    _mm_scale_body,
    grid=(GRID,),
    in_specs=[
        pl.BlockSpec((BM, F), lambda i: (i, 0)),
        pl.BlockSpec((F, F), lambda i: (0, 0)),
        pl.BlockSpec((BM, NC), lambda i: (i, 0)),
    ],
    out_specs=[
        pl.BlockSpec((BM, F), lambda i: (i, 0)),
        pl.BlockSpec((BM, 1), lambda i: (i, 0)),
    ],
    out_shape=[
        jax.ShapeDtypeStruct((N, F), jnp.float32),
        jax.ShapeDtypeStruct((N, 1), jnp.float32),
    ],
)


def _fuse1_body(p_ref, hs1_ref, dinv_ref, b1_ref, w2_ref, h2s_ref):
    dinv = dinv_ref[...]
    t = (p_ref[0] + p_ref[1] + hs1_ref[...]) * dinv + b1_ref[...]
    t = jnp.maximum(t, 0.0)
    h2 = jnp.dot(t, w2_ref[...], preferred_element_type=jnp.float32)
    h2s_ref[...] = h2 * dinv


_fuse1 = pl.pallas_call(
    _fuse1_body,
    grid=(GRID,),
    in_specs=[
        pl.BlockSpec((NC, BM, F), lambda i: (0, i, 0)),
        pl.BlockSpec((BM, F), lambda i: (i, 0)),
        pl.BlockSpec((BM, 1), lambda i: (i, 0)),
        pl.BlockSpec((F,), lambda i: (0,)),
        pl.BlockSpec((F, F), lambda i: (0, 0)),
    ],
    out_specs=pl.BlockSpec((BM, F), lambda i: (i, 0)),
    out_shape=jax.ShapeDtypeStruct((N, F), jnp.float32),
)


def _fuse2_body(q_ref, h2s_ref, dinv_ref, b2_ref, wc_ref, bc_ref, out_ref):
    dinv = dinv_ref[...]
    t = (q_ref[0] + q_ref[1] + h2s_ref[...]) * dinv + b2_ref[...]
    t = jnp.maximum(t, 0.0)
    logits = jnp.dot(t, wc_ref[...], preferred_element_type=jnp.float32)
    logits = logits + bc_ref[...]
    m = jnp.max(logits, axis=1, keepdims=True)
    e = jnp.exp(logits - m)
    lse = jnp.log(jnp.sum(e, axis=1, keepdims=True)) + m
    out_ref[...] = logits - lse


_fuse2 = pl.pallas_call(
    _fuse2_body,
    grid=(GRID,),
    in_specs=[
        pl.BlockSpec((NC, BM, F), lambda i: (0, i, 0)),
        pl.BlockSpec((BM, F), lambda i: (i, 0)),
        pl.BlockSpec((BM, 1), lambda i: (i, 0)),
        pl.BlockSpec((F,), lambda i: (0,)),
        pl.BlockSpec((F, F), lambda i: (0, 0)),
        pl.BlockSpec((F,), lambda i: (0,)),
    ],
    out_specs=pl.BlockSpec((BM, F), lambda i: (i, 0)),
    out_shape=jax.ShapeDtypeStruct((N, F), jnp.float32),
)


@jax.jit
def _run(x, edge_index, W1, b1, W2, b2, Wc, bc):
    src = edge_index[0].astype(jnp.int32)
    dst = edge_index[1].astype(jnp.int32)
    padn = EPAD - E
    ar = jnp.arange(padn, dtype=jnp.int32)
    pad_src = (ar * 131) % N
    pad_dst = N + ar % (NPAD - N)
    srcp = jnp.concatenate([src, pad_src]).reshape(NW, NCHUNK, CHUNK)
    dstp = jnp.concatenate([dst, pad_dst]).reshape(NW, NCHUNK, CHUNK)
    zrows = jnp.zeros((ZR, F), jnp.float32)

    degp = _sc_deg(dstp).reshape(NC, NPAD)[:, :N].T
    hs1, dinv = _mm_scale(x, W1, degp)
    agg1 = _sc_agg(hs1, srcp, dstp, zrows)
    h2s = _fuse1(agg1, hs1, dinv, b1, W2)
    agg2 = _sc_agg(h2s, srcp, dstp, zrows)
    return _fuse2(agg2, h2s, dinv, b2, Wc, bc)


def kernel(x, edge_index, W1, b1, W2, b2, Wc, bc):
    return _run(x, edge_index, W1, b1, W2, b2, Wc, bc)
